# baseline (speedup 1.0000x reference)
import functools

import jax
import jax.numpy as jnp
from jax import lax
from jax.experimental import pallas as pl
from jax.experimental.pallas import tpu as pltpu
from jax.experimental.pallas import tpu_sc as plsc

NC = 2
NS = 16
LANES = 16
CH = 48


def _table_body(x_ref, v_ref, o_ref):
    o_ref[...] = jnp.dot(x_ref[...], v_ref[...],
                         preferred_element_type=jnp.float32)


def _build_table(x, vbig, blk):
    n = x.shape[0]
    return pl.pallas_call(
        _table_body,
        grid=(n // blk,),
        in_specs=[
            pl.BlockSpec((blk, x.shape[1]), lambda i: (i, 0)),
            pl.BlockSpec(vbig.shape, lambda i: (0, 0)),
        ],
        out_specs=pl.BlockSpec((blk, vbig.shape[1]), lambda i: (i, 0)),
        out_shape=jax.ShapeDtypeStruct((n, vbig.shape[1]), jnp.float32),
    )(x, vbig)


def _sc_edge_kernel(n, np_rows, ept, c_half):
    chunks = ept // CH
    rows_per_tile = np_rows // NS
    nr = np_rows // 128
    mesh = plsc.VectorSubcoreMesh(core_axis_name="c", subcore_axis_name="s")

    def body(ty, tz, srcg, dstg, pg, id_tab, out, out_deg, acc, acc_deg,
             idx_v0, idx_v1, dst_v, dstr_v, dstc_v, p_v,
             yrows_v0, yrows_v1, zrows_v0, zrows_v1, oh_v,
             s_gy0, s_gy1, s_gz0, s_gz1, s_sy0, s_sy1, s_sz0, s_sz1,
             s_go, s_ds):
        c = lax.axis_index("c")
        s = lax.axis_index("s")

        zvec = jnp.zeros((LANES,), jnp.float32)

        def zrow(e, _):
            for g in range(c_half // LANES):
                zrows_v0[e, pl.ds(g * LANES, LANES)] = zvec
            return 0
        lax.fori_loop(0, CH, zrow, 0)

        row0 = s * rows_per_tile
        done = 0
        while done < rows_per_tile:
            cnt = min(CH, rows_per_tile - done)
            pltpu.sync_copy(zrows_v0.at[pl.ds(0, cnt)],
                            acc.at[pl.ds(row0 + done, cnt)])
            done += cnt

        @pl.when(s == 0)
        def _():
            dz = 0
            while dz < nr:
                cz = min(CH, nr - dz)
                pltpu.sync_copy(zrows_v0.at[pl.ds(0, cz)],
                                acc_deg.at[pl.ds(dz, cz)])
                dz += cz

        plsc.subcore_barrier()

        pltpu.sync_copy(srcg.at[c, s, 0], idx_v0)
        pltpu.async_copy(ty.at[idx_v0], yrows_v0, s_gy0)
        pltpu.async_copy(tz.at[idx_v0], zrows_v0, s_gz0)

        def chunk(i, _):
            b = lax.rem(i, 2)

            def for_buf(bb, idx_c, y_c, z_c, sgy_c, sgz_c, ssy_c, ssz_c,
                        idx_n, y_n, z_n, sgy_n, sgz_n, ssy_n, ssz_n):
                @pl.when(b == bb)
                def _():
                    @pl.when(i >= 1)
                    def _():
                        pltpu.make_async_copy(y_n, acc.at[dst_v],
                                              ssy_n).wait()
                        pltpu.make_async_copy(z_n, acc.at[dst_v],
                                              ssz_n).wait()

                    @pl.when(i + 1 < chunks)
                    def _():
                        pltpu.sync_copy(srcg.at[c, s, i + 1], idx_n)
                        pltpu.async_copy(ty.at[idx_n], y_n, sgy_n)
                        pltpu.async_copy(tz.at[idx_n], z_n, sgz_n)

                    pltpu.sync_copy(dstg.at[s, i], dst_v)
                    mine = lax.rem(i, NC) == c

                    @pl.when(mine)
                    def _():
                        @pl.when(i >= 2)
                        def _():
                            pltpu.make_async_copy(oh_v,
                                                  acc_deg.at[dstr_v],
                                                  s_ds).wait()
                        for g in range(CH // LANES):
                            v = dst_v[pl.ds(g * LANES, LANES)]
                            dstr_v[pl.ds(g * LANES, LANES)] = v >> 7
                            dstc_v[pl.ds(g * LANES, LANES)] = v & 127
                        pltpu.async_copy(id_tab.at[dstc_v], oh_v, s_go)

                    pltpu.sync_copy(pg.at[s, i], p_v)

                    pltpu.make_async_copy(ty.at[idx_c], y_c, sgy_c).wait()
                    pltpu.async_copy(y_c, acc.at[dst_v], ssy_c, add=True)

                    pltpu.make_async_copy(tz.at[idx_c], z_c, sgz_c).wait()

                    def edge(e, _):
                        p_s = p_v[e, pl.ds(0, LANES)]
                        for g in range(c_half // LANES):
                            za = z_c[e, pl.ds(g * LANES, LANES)]
                            z_c[e, pl.ds(g * LANES, LANES)] = p_s * za
                        return 0
                    lax.fori_loop(0, CH, edge, 0)

                    pltpu.async_copy(z_c, acc.at[dst_v], ssz_c, add=True)

                    @pl.when(mine)
                    def _():
                        pltpu.make_async_copy(id_tab.at[dstc_v], oh_v,
                                              s_go).wait()
                        pltpu.async_copy(oh_v, acc_deg.at[dstr_v], s_ds,
                                         add=True)

            for_buf(0, idx_v0, yrows_v0, zrows_v0, s_gy0, s_gz0, s_sy0,
                    s_sz0, idx_v1, yrows_v1, zrows_v1, s_gy1, s_gz1,
                    s_sy1, s_sz1)
            for_buf(1, idx_v1, yrows_v1, zrows_v1, s_gy1, s_gz1, s_sy1,
                    s_sz1, idx_v0, yrows_v0, zrows_v0, s_gy0, s_gz0,
                    s_sy0, s_sz0)
            return 0
        lax.fori_loop(0, chunks, chunk, 0)

        if (chunks - 1) % 2 == 0:
            pltpu.make_async_copy(yrows_v0, acc.at[dst_v], s_sy0).wait()
            pltpu.make_async_copy(zrows_v0, acc.at[dst_v], s_sz0).wait()
        else:
            pltpu.make_async_copy(yrows_v1, acc.at[dst_v], s_sy1).wait()
            pltpu.make_async_copy(zrows_v1, acc.at[dst_v], s_sz1).wait()
        pltpu.make_async_copy(oh_v, acc_deg.at[dstr_v], s_ds).wait()

        plsc.subcore_barrier()

        pltpu.sync_copy(acc.at[pl.ds(row0, rows_per_tile)],
                        out.at[c, pl.ds(row0, rows_per_tile)])

        @pl.when(s == 0)
        def _():
            pltpu.sync_copy(acc_deg, out_deg.at[c])

    return pl.kernel(
        body,
        out_type=(
            jax.ShapeDtypeStruct((NC, np_rows, c_half), jnp.float32),
            jax.ShapeDtypeStruct((NC, nr, 128), jnp.float32),
        ),
        mesh=mesh,
        scratch_types=[
            pltpu.VMEM_SHARED((np_rows, c_half), jnp.float32),
            pltpu.VMEM_SHARED((nr, 128), jnp.float32),
            pltpu.VMEM((CH,), jnp.int32),
            pltpu.VMEM((CH,), jnp.int32),
            pltpu.VMEM((CH,), jnp.int32),
            pltpu.VMEM((CH,), jnp.int32),
            pltpu.VMEM((CH,), jnp.int32),
            pltpu.VMEM((CH, LANES), jnp.float32),
            pltpu.VMEM((CH, c_half), jnp.float32),
            pltpu.VMEM((CH, c_half), jnp.float32),
            pltpu.VMEM((CH, c_half), jnp.float32),
            pltpu.VMEM((CH, c_half), jnp.float32),
            pltpu.VMEM((CH, 128), jnp.float32),
        ] + [pltpu.SemaphoreType.DMA] * 10,
    )


def _combine_body(acc_ref, deg_ref, x_ref, root_ref, bias_ref, o_ref, *,
                  c_half):
    agg = jnp.concatenate([acc_ref[0], acc_ref[1]], axis=1)
    deg = deg_ref[0] + deg_ref[1]
    xr = jnp.dot(x_ref[...], root_ref[...],
                 preferred_element_type=jnp.float32)
    o_ref[...] = jnp.maximum(
        agg / jnp.maximum(deg, 1.0) + xr + bias_ref[...], 0.0)


def _combine(acc, deg2d, x, root, bias2d, blk, c_half):
    n, c_in = x.shape
    c_out = root.shape[1]
    return pl.pallas_call(
        functools.partial(_combine_body, c_half=c_half),
        grid=(n // blk,),
        in_specs=[
            pl.BlockSpec((NC, blk, c_half), lambda i: (0, i, 0)),
            pl.BlockSpec((NC, blk, 1), lambda i: (0, i, 0)),
            pl.BlockSpec((blk, c_in), lambda i: (i, 0)),
            pl.BlockSpec((c_in, c_out), lambda i: (0, 0)),
            pl.BlockSpec((1, c_out), lambda i: (0, 0)),
        ],
        out_specs=pl.BlockSpec((blk, c_out), lambda i: (i, 0)),
        out_shape=jax.ShapeDtypeStruct((n, c_out), jnp.float32),
    )(acc, deg2d, x, root, bias2d)


def kernel(x, edge_index, edge_attr, weight, root, bias):
    n, c_in = x.shape
    c_out = root.shape[1]
    c_half = c_out // 2
    e = edge_index.shape[1]

    src = edge_index[0]
    dst = edge_index[1]
    p = edge_attr[:, 0]

    epad = ((e + NS * CH - 1) // (NS * CH)) * (NS * CH)
    np_rows = ((n + 1 + 127) // 128) * 128
    pad = epad - e
    src_p = jnp.concatenate([src, jnp.zeros((pad,), jnp.int32)])
    dst_p = jnp.concatenate([dst, jnp.full((pad,), n, jnp.int32)])
    p_p = jnp.concatenate([p, jnp.zeros((pad,), jnp.float32)])
    ept = epad // NS
    chunks = ept // CH
    p16 = jnp.broadcast_to(p_p[:, None], (epad, LANES)
                           ).reshape(NS, chunks, CH, LANES)
    srcg = jnp.stack([2 * src_p, 2 * src_p + 1]
                     ).reshape(NC, NS, chunks, CH)
    dst_r = dst_p.reshape(NS, chunks, CH)

    w0 = weight[0]
    wd = weight[1] - weight[0]
    ty = _build_table(x, w0, blk=2000).reshape(2 * n, c_half)
    tz = _build_table(x, wd, blk=2000).reshape(2 * n, c_half)
    id_tab = jnp.eye(128, dtype=jnp.float32)

    acc, deg = _sc_edge_kernel(n, np_rows, ept, c_half)(
        ty, tz, srcg, dst_r, p16, id_tab)

    deg2d = deg.reshape(NC, np_rows)[:, :n, None]
    return _combine(acc, deg2d, x, root, bias.reshape(1, c_out), blk=2000,
                    c_half=c_half)


# speedup vs baseline: 1.2611x; 1.2611x over previous
import functools

import jax
import jax.numpy as jnp
from jax import lax
from jax.experimental import pallas as pl
from jax.experimental.pallas import tpu as pltpu
from jax.experimental.pallas import tpu_sc as plsc

NC = 2
NS = 16
LANES = 16
CH = 80


def _table_body(x_ref, v_ref, o_ref):
    o_ref[...] = jnp.dot(x_ref[...], v_ref[...],
                         preferred_element_type=jnp.float32)


def _build_table(x, vbig, blk):
    n = x.shape[0]
    return pl.pallas_call(
        _table_body,
        grid=(n // blk,),
        in_specs=[
            pl.BlockSpec((blk, x.shape[1]), lambda i: (i, 0)),
            pl.BlockSpec(vbig.shape, lambda i: (0, 0)),
        ],
        out_specs=pl.BlockSpec((blk, vbig.shape[1]), lambda i: (i, 0)),
        out_shape=jax.ShapeDtypeStruct((n, vbig.shape[1]), jnp.float32),
    )(x, vbig)


def _sc_edge_kernel(n, np_rows, ept, c_half):
    chunks = ept // CH
    rows_per_tile = np_rows // NS
    nr = np_rows // 128
    mesh = plsc.VectorSubcoreMesh(core_axis_name="c", subcore_axis_name="s")

    def body(ty, tz, srcg, dstg, pg, id_tab, out, out_deg, acc, acc_deg,
             idx_v, dst_v, dstr_v, dstc_v, p_v, yrows_v, zrows_v, oh_v,
             s_gy, s_gz, s_sy, s_sz, s_go, s_ds):
        c = lax.axis_index("c")
        s = lax.axis_index("s")

        zvec = jnp.zeros((LANES,), jnp.float32)

        def zrow(e, _):
            for g in range(c_half // LANES):
                zrows_v[e, pl.ds(g * LANES, LANES)] = zvec
            return 0
        lax.fori_loop(0, CH, zrow, 0)

        row0 = s * rows_per_tile
        done = 0
        while done < rows_per_tile:
            cnt = min(CH, rows_per_tile - done)
            pltpu.sync_copy(zrows_v.at[pl.ds(0, cnt)],
                            acc.at[pl.ds(row0 + done, cnt)])
            done += cnt

        @pl.when(s == 0)
        def _():
            dz = 0
            while dz < nr:
                cz = min(CH, nr - dz)
                pltpu.sync_copy(zrows_v.at[pl.ds(0, cz)],
                                acc_deg.at[pl.ds(dz, cz)])
                dz += cz

        plsc.subcore_barrier()

        def chunk(i, _):
            pltpu.sync_copy(srcg.at[c, s, i], idx_v)

            @pl.when(i > 0)
            def _():
                pltpu.make_async_copy(yrows_v, acc.at[dst_v], s_sy).wait()
                pltpu.make_async_copy(zrows_v, acc.at[dst_v], s_sz).wait()

            pltpu.async_copy(ty.at[idx_v], yrows_v, s_gy)
            pltpu.async_copy(tz.at[idx_v], zrows_v, s_gz)

            pltpu.sync_copy(dstg.at[s, i], dst_v)
            mine = lax.rem(i, NC) == c

            @pl.when(mine)
            def _():
                @pl.when(i >= 2)
                def _():
                    pltpu.make_async_copy(oh_v, acc_deg.at[dstr_v],
                                          s_ds).wait()
                for g in range(CH // LANES):
                    v = dst_v[pl.ds(g * LANES, LANES)]
                    dstr_v[pl.ds(g * LANES, LANES)] = v >> 7
                    dstc_v[pl.ds(g * LANES, LANES)] = v & 127
                pltpu.async_copy(id_tab.at[dstc_v], oh_v, s_go)

            pltpu.sync_copy(pg.at[s, i], p_v)

            pltpu.make_async_copy(ty.at[idx_v], yrows_v, s_gy).wait()
            pltpu.async_copy(yrows_v, acc.at[dst_v], s_sy, add=True)

            pltpu.make_async_copy(tz.at[idx_v], zrows_v, s_gz).wait()

            def edge(e, _):
                p_s = p_v[e, pl.ds(0, LANES)]
                for g in range(c_half // LANES):
                    za = zrows_v[e, pl.ds(g * LANES, LANES)]
                    zrows_v[e, pl.ds(g * LANES, LANES)] = p_s * za
                return 0
            lax.fori_loop(0, CH, edge, 0)

            pltpu.async_copy(zrows_v, acc.at[dst_v], s_sz, add=True)

            @pl.when(mine)
            def _():
                pltpu.make_async_copy(id_tab.at[dstc_v], oh_v, s_go).wait()
                pltpu.async_copy(oh_v, acc_deg.at[dstr_v], s_ds, add=True)
            return 0
        lax.fori_loop(0, chunks, chunk, 0)

        pltpu.make_async_copy(yrows_v, acc.at[dst_v], s_sy).wait()
        pltpu.make_async_copy(zrows_v, acc.at[dst_v], s_sz).wait()
        pltpu.make_async_copy(oh_v, acc_deg.at[dstr_v], s_ds).wait()

        plsc.subcore_barrier()

        pltpu.sync_copy(acc.at[pl.ds(row0, rows_per_tile)],
                        out.at[c, pl.ds(row0, rows_per_tile)])

        @pl.when(s == 0)
        def _():
            pltpu.sync_copy(acc_deg, out_deg.at[c])

    return pl.kernel(
        body,
        out_type=(
            jax.ShapeDtypeStruct((NC, np_rows, c_half), jnp.float32),
            jax.ShapeDtypeStruct((NC, nr, 128), jnp.float32),
        ),
        mesh=mesh,
        scratch_types=[
            pltpu.VMEM_SHARED((np_rows, c_half), jnp.float32),
            pltpu.VMEM_SHARED((nr, 128), jnp.float32),
            pltpu.VMEM((CH,), jnp.int32),
            pltpu.VMEM((CH,), jnp.int32),
            pltpu.VMEM((CH,), jnp.int32),
            pltpu.VMEM((CH,), jnp.int32),
            pltpu.VMEM((CH, LANES), jnp.float32),
            pltpu.VMEM((CH, c_half), jnp.float32),
            pltpu.VMEM((CH, c_half), jnp.float32),
            pltpu.VMEM((CH, 128), jnp.float32),
            pltpu.SemaphoreType.DMA,
            pltpu.SemaphoreType.DMA,
            pltpu.SemaphoreType.DMA,
            pltpu.SemaphoreType.DMA,
            pltpu.SemaphoreType.DMA,
            pltpu.SemaphoreType.DMA,
        ],
    )


def _combine_body(acc_ref, deg_ref, x_ref, root_ref, bias_ref, o_ref, *,
                  c_half):
    agg = jnp.concatenate([acc_ref[0], acc_ref[1]], axis=1)
    deg = deg_ref[0] + deg_ref[1]
    xr = jnp.dot(x_ref[...], root_ref[...],
                 preferred_element_type=jnp.float32)
    o_ref[...] = jnp.maximum(
        agg / jnp.maximum(deg, 1.0) + xr + bias_ref[...], 0.0)


def _combine(acc, deg2d, x, root, bias2d, blk, c_half):
    n, c_in = x.shape
    c_out = root.shape[1]
    return pl.pallas_call(
        functools.partial(_combine_body, c_half=c_half),
        grid=(n // blk,),
        in_specs=[
            pl.BlockSpec((NC, blk, c_half), lambda i: (0, i, 0)),
            pl.BlockSpec((NC, blk, 1), lambda i: (0, i, 0)),
            pl.BlockSpec((blk, c_in), lambda i: (i, 0)),
            pl.BlockSpec((c_in, c_out), lambda i: (0, 0)),
            pl.BlockSpec((1, c_out), lambda i: (0, 0)),
        ],
        out_specs=pl.BlockSpec((blk, c_out), lambda i: (i, 0)),
        out_shape=jax.ShapeDtypeStruct((n, c_out), jnp.float32),
    )(acc, deg2d, x, root, bias2d)


def kernel(x, edge_index, edge_attr, weight, root, bias):
    n, c_in = x.shape
    c_out = root.shape[1]
    c_half = c_out // 2
    e = edge_index.shape[1]

    src = edge_index[0]
    dst = edge_index[1]
    p = edge_attr[:, 0]

    epad = ((e + NS * CH - 1) // (NS * CH)) * (NS * CH)
    np_rows = ((n + 1 + 127) // 128) * 128
    pad = epad - e
    src_p = jnp.concatenate([src, jnp.zeros((pad,), jnp.int32)])
    dst_p = jnp.concatenate([dst, jnp.full((pad,), n, jnp.int32)])
    p_p = jnp.concatenate([p, jnp.zeros((pad,), jnp.float32)])
    ept = epad // NS
    chunks = ept // CH
    p16 = jnp.broadcast_to(p_p[:, None], (epad, LANES)
                           ).reshape(NS, chunks, CH, LANES)
    srcg = jnp.stack([2 * src_p, 2 * src_p + 1]
                     ).reshape(NC, NS, chunks, CH)
    dst_r = dst_p.reshape(NS, chunks, CH)

    w0 = weight[0]
    wd = weight[1] - weight[0]
    ty = _build_table(x, w0, blk=2000).reshape(2 * n, c_half)
    tz = _build_table(x, wd, blk=2000).reshape(2 * n, c_half)
    id_tab = jnp.eye(128, dtype=jnp.float32)

    acc, deg = _sc_edge_kernel(n, np_rows, ept, c_half)(
        ty, tz, srcg, dst_r, p16, id_tab)

    deg2d = deg.reshape(NC, np_rows)[:, :n, None]
    return _combine(acc, deg2d, x, root, bias.reshape(1, c_out), blk=2000,
                    c_half=c_half)
